# baseline (speedup 1.0000x reference)
import functools

import jax
import jax.numpy as jnp
from jax import lax
from jax.experimental import pallas as pl
from jax.experimental.pallas import tpu as pltpu
from jax.experimental.pallas import tpu_sc as plsc

NC = 2
NS = 16
NW = NC * NS
CHUNK = 128


@functools.lru_cache(maxsize=None)
def _make_gather(V, D, B):
    assert B % (NW * CHUNK) == 0
    b_per_w = B // NW
    n_chunks = b_per_w // CHUNK
    mesh = plsc.VectorSubcoreMesh(core_axis_name="c", subcore_axis_name="s")

    @functools.partial(
        pl.kernel,
        out_type=jax.ShapeDtypeStruct((B, D), jnp.float32),
        mesh=mesh,
        scratch_types=[
            pltpu.VMEM((n_chunks, CHUNK), jnp.int32),
            pltpu.VMEM((CHUNK, D), jnp.float32),
            pltpu.VMEM((CHUNK, D), jnp.float32),
            pltpu.SemaphoreType.DMA,
            pltpu.SemaphoreType.DMA,
            pltpu.SemaphoreType.DMA,
            pltpu.SemaphoreType.DMA,
        ],
        compiler_params=pltpu.CompilerParams(use_tc_tiling_on_sc=False),
    )
    def gather_kernel(table_hbm, idx_hbm, out_hbm, idx_v, rows0, rows1,
                      gsem0, gsem1, ssem0, ssem1):
        wid = lax.axis_index("s") * NC + lax.axis_index("c")
        base = wid * b_per_w
        pltpu.sync_copy(idx_hbm.at[wid], idx_v)

        @pl.loop(0, n_chunks, step=2)
        def _(g):
            g0 = pltpu.async_copy(table_hbm.at[idx_v.at[g]], rows0, gsem0)
            g1 = pltpu.async_copy(table_hbm.at[idx_v.at[g + 1]], rows1, gsem1)
            g0.wait()
            s0 = pltpu.async_copy(
                rows0, out_hbm.at[pl.ds(base + g * CHUNK, CHUNK)], ssem0)
            g1.wait()
            s1 = pltpu.async_copy(
                rows1, out_hbm.at[pl.ds(base + (g + 1) * CHUNK, CHUNK)], ssem1)
            s0.wait()
            s1.wait()

    return gather_kernel


def kernel(x, pe_weight):
    Bt, H = x.shape
    V, D = pe_weight.shape
    B = Bt * H
    idx = x.reshape(NW, B // (NW * CHUNK), CHUNK).astype(jnp.int32)
    out = _make_gather(V, D, B)(pe_weight, idx)
    return out.reshape(Bt, H, D)


# speedup vs baseline: 1.0342x; 1.0342x over previous
import functools

import jax
import jax.numpy as jnp
from jax import lax
from jax.experimental import pallas as pl
from jax.experimental.pallas import tpu as pltpu
from jax.experimental.pallas import tpu_sc as plsc

NC = 2
NS = 16
NW = NC * NS
CHUNK = 128


SUB = 4
SUPER = SUB * CHUNK


@functools.lru_cache(maxsize=None)
def _make_gather(V, D, B):
    assert B % (NW * SUPER) == 0
    b_per_w = B // NW
    n_chunks = b_per_w // CHUNK
    n_super = b_per_w // SUPER
    mesh = plsc.VectorSubcoreMesh(core_axis_name="c", subcore_axis_name="s")

    @functools.partial(
        pl.kernel,
        out_type=jax.ShapeDtypeStruct((B, D), jnp.float32),
        mesh=mesh,
        scratch_types=[
            pltpu.VMEM((n_chunks, CHUNK), jnp.int32),
            pltpu.VMEM((SUPER, D), jnp.float32),
            pltpu.VMEM((SUPER, D), jnp.float32),
            pltpu.SemaphoreType.DMA,
            pltpu.SemaphoreType.DMA,
            pltpu.SemaphoreType.DMA,
            pltpu.SemaphoreType.DMA,
        ],
        compiler_params=pltpu.CompilerParams(use_tc_tiling_on_sc=False),
    )
    def gather_kernel(table_hbm, idx_hbm, out_hbm, idx_v, rows0, rows1,
                      gsem0, gsem1, ssem0, ssem1):
        wid = lax.axis_index("s") * NC + lax.axis_index("c")
        base = wid * b_per_w
        pltpu.sync_copy(idx_hbm.at[wid], idx_v)

        def fire_gathers(s, rows, gsem):
            cps = []
            for j in range(SUB):
                cps.append(pltpu.async_copy(
                    table_hbm.at[idx_v.at[s * SUB + j]],
                    rows.at[pl.ds(j * CHUNK, CHUNK)], gsem))
            return cps

        @pl.loop(0, n_super, step=2)
        def _(s):
            g0 = fire_gathers(s, rows0, gsem0)
            g1 = fire_gathers(s + 1, rows1, gsem1)
            for cp in g0:
                cp.wait()
            s0 = pltpu.async_copy(
                rows0, out_hbm.at[pl.ds(base + s * SUPER, SUPER)], ssem0)
            for cp in g1:
                cp.wait()
            s1 = pltpu.async_copy(
                rows1, out_hbm.at[pl.ds(base + (s + 1) * SUPER, SUPER)], ssem1)
            s0.wait()
            s1.wait()

    return gather_kernel


def kernel(x, pe_weight):
    Bt, H = x.shape
    V, D = pe_weight.shape
    B = Bt * H
    idx = x.reshape(NW, B // (NW * CHUNK), CHUNK).astype(jnp.int32)
    out = _make_gather(V, D, B)(pe_weight, idx)
    return out.reshape(Bt, H, D)
